# baseline (speedup 1.0000x reference)
import functools

import numpy as np
import jax
import jax.numpy as jnp
from jax import lax
from jax.experimental import pallas as pl
from jax.experimental.pallas import tpu as pltpu
from jax.experimental.pallas import tpu_sc as plsc

NC = 2
NS = 16
NW = NC * NS
K = 16
NB = 2
LANES = 16


def _pos_enc_np(seq_len, d_model):
    pos = np.arange(seq_len, dtype=np.float32)[:, None]
    _2i = np.arange(0, d_model, 2, dtype=np.float32)
    angle = pos / np.power(10000.0, _2i / np.float32(d_model))
    enc = np.zeros((seq_len, d_model), dtype=np.float32)
    enc[:, 0::2] = np.sin(angle)
    enc[:, 1::2] = np.cos(angle)
    return enc


def _pos_packed(seq_len, d_model):
    import ml_dtypes
    enc = _pos_enc_np(seq_len, d_model)
    b16 = enc.astype(ml_dtypes.bfloat16).view(np.uint16)
    g = b16.reshape(seq_len, d_model // 32, 2, 16)
    words = g[:, :, 0, :].astype(np.uint32) | (
        g[:, :, 1, :].astype(np.uint32) << 16)
    return jnp.asarray(
        words.reshape(seq_len * d_model // 2).view(np.float32))


def kernel(x, table):
    b, s = x.shape
    v, d = table.shape
    ppw = s // NW
    npb = ppw // K
    n2 = d // 32
    shift = n2.bit_length() - 1
    assert n2 == 1 << shift and ppw == npb * K

    pos = _pos_packed(s, d)

    mesh = plsc.VectorSubcoreMesh(core_axis_name="c", subcore_axis_name="s")

    @functools.partial(
        pl.kernel,
        mesh=mesh,
        out_type=jax.ShapeDtypeStruct((b, s, d), jnp.float32),
        scratch_types=[
            pltpu.VMEM((b, ppw), jnp.int32),
            pltpu.VMEM((K * d // 2,), jnp.float32),
            pltpu.VMEM((K * d // 2,), jnp.float32),
            pltpu.VMEM((K, d), jnp.float32),
            pltpu.VMEM((K, d), jnp.float32),
            pltpu.SemaphoreType.DMA,
            pltpu.SemaphoreType.DMA,
            pltpu.SemaphoreType.DMA,
            pltpu.SemaphoreType.DMA,
            pltpu.SemaphoreType.DMA,
            pltpu.SemaphoreType.DMA,
            pltpu.SemaphoreType.DMA,
        ],
    )
    def emb(x_hbm, table_hbm, pos_hbm, out_hbm,
            idx_v, pv0, pv1, r0, r1,
            g0, g1, w0, w1, ps0, ps1, xs):
        wid = lax.axis_index("s") * NC + lax.axis_index("c")
        pbase = wid * ppw

        def pos_load(q, buf, sem):
            off = pl.multiple_of((pbase + q * K) * (d // 2), 8)
            return pltpu.async_copy(
                pos_hbm.at[pl.ds(off, K * d // 2)], buf, sem)

        posb = [pv0, pv1]
        psem = [ps0, ps1]
        pld = [pos_load(0, posb[0], psem[0]), None]
        xld = [pltpu.async_copy(
                   x_hbm.at[bb, pl.ds(pbase, ppw)], idx_v.at[bb], xs)
               for bb in range(b)]
        for cp in xld:
            cp.wait()

        rows = [r0, r1]
        gsem = [g0, g1]
        wsem = [w0, w1]
        chunks = [(q, bb) for q in range(npb) for bb in range(b)]
        n = len(chunks)

        def gather(i, buf, sem):
            q, bb = chunks[i]
            return pltpu.async_copy(
                table_hbm.at[idx_v.at[bb, pl.ds(q * K, K)]], buf, sem)

        gat = [gather(0, rows[0], gsem[0]), None]
        wr = [None, None]

        for i, (q, bb) in enumerate(chunks):
            p = i % NB
            pq = q & 1
            if i + 1 < n:
                t = (i + 1) % NB
                if wr[t] is not None:
                    wr[t].wait()
                gat[t] = gather(i + 1, rows[t], gsem[t])
            if bb == 0:
                if q + 1 < npb:
                    pld[1 - pq] = pos_load(q + 1, posb[1 - pq], psem[1 - pq])
                if pld[pq] is not None:
                    pld[pq].wait()
                    pld[pq] = None
            gat[p].wait()

            rbuf = rows[p]
            pbuf = posb[pq]

            @plsc.parallel_loop(0, K * n2, 1, unroll=8)
            def _(it):
                r = it >> shift
                c = it & (n2 - 1)
                u = pbuf[pl.ds(pl.multiple_of(it * LANES, 8), LANES)]
                ui = lax.bitcast_convert_type(u, jnp.int32)
                pa = lax.bitcast_convert_type(
                    lax.shift_left(ui, 16), jnp.float32)
                pb = lax.bitcast_convert_type(
                    lax.bitwise_and(ui, jnp.int32(-65536)), jnp.float32)
                plsc.addupdate(rbuf.at[r, pl.ds(c * 32, LANES)], pa)
                plsc.addupdate(rbuf.at[r, pl.ds(c * 32 + LANES, LANES)], pb)

            wr[p] = pltpu.async_copy(
                rbuf, out_hbm.at[bb, pl.ds(pbase + q * K, K)], wsem[p])

        for p in range(NB):
            if wr[p] is not None:
                wr[p].wait()

    return emb(x, table, pos)


# speedup vs baseline: 1.0072x; 1.0072x over previous
import functools

import numpy as np
import jax
import jax.numpy as jnp
from jax import lax
from jax.experimental import pallas as pl
from jax.experimental.pallas import tpu as pltpu
from jax.experimental.pallas import tpu_sc as plsc

NC = 2
NS = 16
NW = NC * NS
K = 16
NB = 3
LANES = 16


def _pos_enc_np(seq_len, d_model):
    pos = np.arange(seq_len, dtype=np.float32)[:, None]
    _2i = np.arange(0, d_model, 2, dtype=np.float32)
    angle = pos / np.power(10000.0, _2i / np.float32(d_model))
    enc = np.zeros((seq_len, d_model), dtype=np.float32)
    enc[:, 0::2] = np.sin(angle)
    enc[:, 1::2] = np.cos(angle)
    return enc


def _pos_packed(seq_len, d_model):
    import ml_dtypes
    enc = _pos_enc_np(seq_len, d_model)
    b16 = enc.astype(ml_dtypes.bfloat16).view(np.uint16)
    g = b16.reshape(seq_len, d_model // 32, 2, 16)
    words = g[:, :, 0, :].astype(np.uint32) | (
        g[:, :, 1, :].astype(np.uint32) << 16)
    return jnp.asarray(
        words.reshape(seq_len * d_model // 2).view(np.float32))


def kernel(x, table):
    b, s = x.shape
    v, d = table.shape
    ppw = s // NW
    npb = ppw // K
    n2 = d // 32
    shift = n2.bit_length() - 1
    assert n2 == 1 << shift and ppw == npb * K

    pos = _pos_packed(s, d)

    mesh = plsc.VectorSubcoreMesh(core_axis_name="c", subcore_axis_name="s")

    @functools.partial(
        pl.kernel,
        mesh=mesh,
        out_type=jax.ShapeDtypeStruct((b, s, d), jnp.float32),
        scratch_types=[
            pltpu.VMEM((b, ppw), jnp.int32),
            pltpu.VMEM((K * d // 2,), jnp.float32),
            pltpu.VMEM((K, d), jnp.float32),
            pltpu.VMEM((K, d), jnp.float32),
            pltpu.VMEM((K, d), jnp.float32),
            pltpu.SemaphoreType.DMA,
            pltpu.SemaphoreType.DMA,
            pltpu.SemaphoreType.DMA,
            pltpu.SemaphoreType.DMA,
            pltpu.SemaphoreType.DMA,
            pltpu.SemaphoreType.DMA,
            pltpu.SemaphoreType.DMA,
            pltpu.SemaphoreType.DMA,
        ],
    )
    def emb(x_hbm, table_hbm, pos_hbm, out_hbm,
            idx_v, pv0, r0, r1, r2,
            g0, g1, g2, w0, w1, w2, ps0, xs):
        wid = lax.axis_index("s") * NC + lax.axis_index("c")
        pbase = wid * ppw

        def pos_load(q, buf, sem):
            off = pl.multiple_of((pbase + q * K) * (d // 2), 8)
            return pltpu.async_copy(
                pos_hbm.at[pl.ds(off, K * d // 2)], buf, sem)

        pld = [pos_load(0, pv0, ps0)]
        xld = [pltpu.async_copy(
                   x_hbm.at[bb, pl.ds(pbase, ppw)], idx_v.at[bb], xs)
               for bb in range(b)]
        for cp in xld:
            cp.wait()

        rows = [r0, r1, r2]
        gsem = [g0, g1, g2]
        wsem = [w0, w1, w2]
        chunks = [(q, bb) for q in range(npb) for bb in range(b)]
        n = len(chunks)

        def gather(i, buf, sem):
            q, bb = chunks[i]
            return pltpu.async_copy(
                table_hbm.at[idx_v.at[bb, pl.ds(q * K, K)]], buf, sem)

        gat = [gather(0, rows[0], gsem[0]),
               gather(1, rows[1], gsem[1]), None]
        wr = [None, None, None]

        for i, (q, bb) in enumerate(chunks):
            p = i % NB
            if i + 2 < n:
                t = (i + 2) % NB
                if wr[t] is not None:
                    wr[t].wait()
                gat[t] = gather(i + 2, rows[t], gsem[t])
            if bb == 0 and pld[0] is not None:
                pld[0].wait()
                pld[0] = None
            gat[p].wait()

            rbuf = rows[p]
            pbuf = pv0

            @plsc.parallel_loop(0, K * n2, 1, unroll=4)
            def _(it):
                r = it >> shift
                c = it & (n2 - 1)
                u = pbuf[pl.ds(pl.multiple_of(it * LANES, 8), LANES)]
                ui = lax.bitcast_convert_type(u, jnp.int32)
                pa = lax.bitcast_convert_type(
                    lax.shift_left(ui, 16), jnp.float32)
                pb = lax.bitcast_convert_type(
                    lax.bitwise_and(ui, jnp.int32(-65536)), jnp.float32)
                plsc.addupdate(rbuf.at[r, pl.ds(c * 32, LANES)], pa)
                plsc.addupdate(rbuf.at[r, pl.ds(c * 32 + LANES, LANES)], pb)

            wr[p] = pltpu.async_copy(
                rbuf, out_hbm.at[bb, pl.ds(pbase + q * K, K)], wsem[p])
            if bb == b - 1 and q + 1 < npb:
                pld[0] = pos_load(q + 1, pv0, ps0)

        for p in range(NB):
            if wr[p] is not None:
                wr[p].wait()

    return emb(x, table, pos)
